# baseline (device time: 6544 ns/iter reference)
import jax
import jax.numpy as jnp
from jax import lax
from jax.experimental import pallas as pl
from jax.experimental.pallas import tpu as pltpu

N_DEV = 4


def kernel(x):
    m, n = x.shape

    def body(x_ref, out_ref, top_ref, bot_ref, send_sems, recv_sems):
        my_pos = lax.axis_index("i")
        left = (my_pos - 1) % N_DEV
        right = (my_pos + 1) % N_DEV

        barrier_sem = pltpu.get_barrier_semaphore()
        for nbr in [left, right]:
            pl.semaphore_signal(
                barrier_sem, inc=1,
                device_id=(nbr,), device_id_type=pl.DeviceIdType.MESH,
            )
        pl.semaphore_wait(barrier_sem, 2)

        rdma_r = pltpu.make_async_remote_copy(
            src_ref=x_ref.at[pl.ds(m - 1, 1)],
            dst_ref=top_ref,
            send_sem=send_sems.at[0],
            recv_sem=recv_sems.at[0],
            device_id=(right,),
            device_id_type=pl.DeviceIdType.MESH,
        )
        rdma_l = pltpu.make_async_remote_copy(
            src_ref=x_ref.at[pl.ds(0, 1)],
            dst_ref=bot_ref,
            send_sem=send_sems.at[1],
            recv_sem=recv_sems.at[1],
            device_id=(left,),
            device_id_type=pl.DeviceIdType.MESH,
        )
        rdma_r.start()
        rdma_l.start()

        x = x_ref[...]
        out_ref[pl.ds(1, m - 2), :] = (
            0.25 * x[: m - 2] + 0.5 * x[1 : m - 1] + 0.25 * x[2:]
        )

        rdma_r.wait_recv()
        rdma_l.wait_recv()

        top_out = 0.25 * top_ref[...] + 0.5 * x[0:1] + 0.25 * x[1:2]
        out_ref[pl.ds(0, 1), :] = jnp.where(my_pos == 0, x[0:1], top_out)
        bot_out = 0.25 * x[m - 2 : m - 1] + 0.5 * x[m - 1 :] + 0.25 * bot_ref[...]
        out_ref[pl.ds(m - 1, 1), :] = jnp.where(
            my_pos == N_DEV - 1, x[m - 1 :], bot_out
        )

        rdma_r.wait_send()
        rdma_l.wait_send()

    return pl.pallas_call(
        body,
        out_shape=jax.ShapeDtypeStruct((m, n), x.dtype),
        in_specs=[pl.BlockSpec(memory_space=pltpu.VMEM)],
        out_specs=pl.BlockSpec(memory_space=pltpu.VMEM),
        scratch_shapes=[
            pltpu.VMEM((1, n), x.dtype),
            pltpu.VMEM((1, n), x.dtype),
            pltpu.SemaphoreType.DMA((2,)),
            pltpu.SemaphoreType.DMA((2,)),
        ],
        compiler_params=pltpu.CompilerParams(collective_id=0),
    )(x)


# device time: 6523 ns/iter; 1.0032x vs baseline; 1.0032x over previous
import jax
import jax.numpy as jnp
from jax import lax
from jax.experimental import pallas as pl
from jax.experimental.pallas import tpu as pltpu

N_DEV = 4


def kernel(x):
    m, n = x.shape

    def body(x_ref, out_ref, top_ref, bot_ref, send_sems, recv_sems):
        my_pos = lax.axis_index("i")
        left = (my_pos - 1) % N_DEV
        right = (my_pos + 1) % N_DEV

        barrier_sem = pltpu.get_barrier_semaphore()
        for nbr in [left, right]:
            pl.semaphore_signal(
                barrier_sem, inc=1,
                device_id=(nbr,), device_id_type=pl.DeviceIdType.MESH,
            )
        pl.semaphore_wait(barrier_sem, 2)

        rdma_r = pltpu.make_async_remote_copy(
            src_ref=x_ref.at[pl.ds(m - 1, 1)],
            dst_ref=top_ref,
            send_sem=send_sems.at[0],
            recv_sem=recv_sems.at[0],
            device_id=(right,),
            device_id_type=pl.DeviceIdType.MESH,
        )
        rdma_l = pltpu.make_async_remote_copy(
            src_ref=x_ref.at[pl.ds(0, 1)],
            dst_ref=bot_ref,
            send_sem=send_sems.at[1],
            recv_sem=recv_sems.at[1],
            device_id=(left,),
            device_id_type=pl.DeviceIdType.MESH,
        )
        rdma_r.start()
        rdma_l.start()

        x = x_ref[...]
        prev_l = jnp.concatenate([x[0:1], x[: m - 1]], axis=0)
        next_l = jnp.concatenate([x[1:], x[m - 1 :]], axis=0)
        out_ref[...] = 0.25 * prev_l + 0.5 * x + 0.25 * next_l

        rdma_r.wait_recv()
        rdma_l.wait_recv()

        top_out = 0.25 * top_ref[...] + 0.5 * x[0:1] + 0.25 * x[1:2]
        out_ref[pl.ds(0, 1), :] = jnp.where(my_pos == 0, x[0:1], top_out)
        bot_out = 0.25 * x[m - 2 : m - 1] + 0.5 * x[m - 1 :] + 0.25 * bot_ref[...]
        out_ref[pl.ds(m - 1, 1), :] = jnp.where(
            my_pos == N_DEV - 1, x[m - 1 :], bot_out
        )

        rdma_r.wait_send()
        rdma_l.wait_send()

    return pl.pallas_call(
        body,
        out_shape=jax.ShapeDtypeStruct((m, n), x.dtype),
        in_specs=[pl.BlockSpec(memory_space=pltpu.VMEM)],
        out_specs=pl.BlockSpec(memory_space=pltpu.VMEM),
        scratch_shapes=[
            pltpu.VMEM((1, n), x.dtype),
            pltpu.VMEM((1, n), x.dtype),
            pltpu.SemaphoreType.DMA((2,)),
            pltpu.SemaphoreType.DMA((2,)),
        ],
        compiler_params=pltpu.CompilerParams(collective_id=0),
    )(x)


# device time: 6500 ns/iter; 1.0068x vs baseline; 1.0035x over previous
import jax
import jax.numpy as jnp
from jax import lax
from jax.experimental import pallas as pl
from jax.experimental.pallas import tpu as pltpu

N_DEV = 4


def kernel(x):
    m, n = x.shape

    def body(x_ref, out_ref, top_ref, bot_ref, send_sems, recv_sems):
        my_pos = lax.axis_index("i")
        left = (my_pos - 1) % N_DEV
        right = (my_pos + 1) % N_DEV

        barrier_sem = pltpu.get_barrier_semaphore()
        for nbr in [left, right]:
            pl.semaphore_signal(
                barrier_sem, inc=1,
                device_id=(nbr,), device_id_type=pl.DeviceIdType.MESH,
            )
        pl.semaphore_wait(barrier_sem, 2)

        rdma_r = pltpu.make_async_remote_copy(
            src_ref=x_ref.at[pl.ds(m - 1, 1)],
            dst_ref=top_ref,
            send_sem=send_sems.at[0],
            recv_sem=recv_sems.at[0],
            device_id=(right,),
            device_id_type=pl.DeviceIdType.MESH,
        )
        rdma_l = pltpu.make_async_remote_copy(
            src_ref=x_ref.at[pl.ds(0, 1)],
            dst_ref=bot_ref,
            send_sem=send_sems.at[1],
            recv_sem=recv_sems.at[1],
            device_id=(left,),
            device_id_type=pl.DeviceIdType.MESH,
        )
        rdma_r.start()
        rdma_l.start()

        x = x_ref[...]
        prev_l = jnp.concatenate([x[0:1], x[: m - 1]], axis=0)
        next_l = jnp.concatenate([x[1:], x[m - 1 :]], axis=0)
        out_ref[...] = 0.25 * prev_l + 0.5 * x + 0.25 * next_l
        top_local = 0.5 * x[0:1] + 0.25 * x[1:2]
        bot_local = 0.25 * x[m - 2 : m - 1] + 0.5 * x[m - 1 :]

        rdma_r.wait_recv()
        rdma_l.wait_recv()

        out_ref[pl.ds(0, 1), :] = jnp.where(
            my_pos == 0, x[0:1], top_local + 0.25 * top_ref[...]
        )
        out_ref[pl.ds(m - 1, 1), :] = jnp.where(
            my_pos == N_DEV - 1, x[m - 1 :], bot_local + 0.25 * bot_ref[...]
        )

        rdma_r.wait_send()
        rdma_l.wait_send()

    return pl.pallas_call(
        body,
        out_shape=jax.ShapeDtypeStruct((m, n), x.dtype),
        in_specs=[pl.BlockSpec(memory_space=pltpu.VMEM)],
        out_specs=pl.BlockSpec(memory_space=pltpu.VMEM),
        scratch_shapes=[
            pltpu.VMEM((1, n), x.dtype),
            pltpu.VMEM((1, n), x.dtype),
            pltpu.SemaphoreType.DMA((2,)),
            pltpu.SemaphoreType.DMA((2,)),
        ],
        compiler_params=pltpu.CompilerParams(collective_id=0),
    )(x)
